# Initial kernel scaffold; baseline (speedup 1.0000x reference)
import jax
import jax.numpy as jnp
from jax.experimental import pallas as pl


def kernel(x, y, mask):
    raise NotImplementedError("write your pallas kernel here")


# speedup vs baseline: 33.5755x; 33.5755x over previous
import jax
import jax.numpy as jnp
from jax import lax
from jax.experimental import pallas as pl

B, C, N = 16, 128, 512
BIG = 1e30


def _chamfer_kernel(x_ref, y_ref, m_ref, out_ref):
    b = pl.program_id(0)
    xb = x_ref[0]
    yb = y_ref[0]
    m = m_ref[0]

    g = lax.dot_general(
        xb, yb,
        dimension_numbers=(((0,), (0,)), ((), ())),
        preferred_element_type=jnp.float32,
    )

    xsq = jnp.sum(xb * xb, axis=0)[:, None]
    ysq = jnp.sum(yb * yb, axis=0)[None, :]
    l2 = xsq + jnp.reshape(ysq, (1, N)) - 2.0 * g

    m_row = m
    m_col = jnp.reshape(m, (N, 1))
    valid = m_col * m_row
    l2m = jnp.where(valid > 0.5, l2, BIG)

    x_min = jnp.min(l2m, axis=0, keepdims=True)
    y_min = jnp.min(l2m, axis=1, keepdims=True)
    x_dist = jnp.sum(jnp.where(m_row > 0.5, x_min, 0.0))
    y_dist = jnp.sum(jnp.where(m_col > 0.5, y_min, 0.0))
    dist = (x_dist + y_dist) * jnp.float32(1.0 / B)

    @pl.when(b == 0)
    def _init():
        out_ref[...] = jnp.zeros((1, 1), jnp.float32)

    out_ref[...] += jnp.reshape(dist, (1, 1))


def kernel(x, y, mask):
    mf = mask.astype(jnp.float32).reshape(B, 1, N)
    out = pl.pallas_call(
        _chamfer_kernel,
        grid=(B,),
        in_specs=[
            pl.BlockSpec((1, C, N), lambda b: (b, 0, 0)),
            pl.BlockSpec((1, C, N), lambda b: (b, 0, 0)),
            pl.BlockSpec((1, 1, N), lambda b: (b, 0, 0)),
        ],
        out_specs=pl.BlockSpec((1, 1), lambda b: (0, 0)),
        out_shape=jax.ShapeDtypeStruct((1, 1), jnp.float32),
    )(x, y, mf)
    return out[0, 0]
